# baseline (speedup 1.0000x reference)
import jax
import jax.numpy as jnp
from jax import lax
from jax.experimental import pallas as pl
from jax.experimental.pallas import tpu as pltpu
import jax.experimental.pallas.tpu_sc as plsc

N = 10000
E = 320000
T = 2
L = 2
H = 128
O = 64
R = 5
G = 32
BN_EPS = 1e-5

BLK = 1000
NBLK = N // BLK

NC = 2
NS = 16
HH = H // NC
EPT = E // NS
K = 80
EPTP = 20480
CHUNKS = EPTP // K
NBUF = 8
GLEAD = 4
NP = 10240
RPT = NP // NS


def _prep_body(src_ref, ea_ref, idx0_ref, idx1_ref):
    base = (ea_ref[...] * N + src_ref[...]) * NC
    idx0_ref[...] = base
    idx1_ref[...] = base + 1


def _stage_a1_body(preh_ref, scale_ref, shift_ref, vcat_ref, hv_ref):
    hn = preh_ref[...] * scale_ref[...] + shift_ref[...]
    res = lax.dot_general(hn.astype(jnp.bfloat16), vcat_ref[...],
                          (((1,), (0,)), ((), ())),
                          preferred_element_type=jnp.float32)
    for r in range(R):
        hv_ref[r] = res[:, r * H:(r + 1) * H]


def _stage_a2_body(preh_ref, scale_ref, shift_ref, a_ref, b_ref, bp_ref,
                   ha_ref, pooled_ref):
    i = pl.program_id(0)
    hn = preh_ref[...] * scale_ref[...] + shift_ref[...]
    ha_ref[...] = lax.dot_general(hn.astype(jnp.bfloat16), a_ref[...],
                                  (((1,), (0,)), ((), ())),
                                  preferred_element_type=jnp.float32) + b_ref[...]
    part = lax.dot_general(bp_ref[...], hn, (((0,), (0,)), ((), ())),
                           preferred_element_type=jnp.float32,
                           precision=lax.Precision.HIGHEST)

    @pl.when(i == 0)
    def _():
        pooled_ref[...] = part

    @pl.when(i > 0)
    def _():
        pooled_ref[...] += part


def _stage_c_body(ha_ref, agg_ref, pooled_ref, rw_ref, bp_ref, g_ref, be_ref,
                  preh_ref, scale_ref, shift_ref, acc_ref):
    i = pl.program_id(0)
    rd = lax.dot_general(pooled_ref[...].astype(jnp.bfloat16), rw_ref[...],
                         (((1,), (0,)), ((), ())),
                         preferred_element_type=jnp.float32)
    agg_full = jnp.concatenate([agg_ref[0], agg_ref[1]], axis=1)
    out = (ha_ref[...] + agg_full
           + lax.dot_general(bp_ref[...], rd, (((1,), (0,)), ((), ())),
                             preferred_element_type=jnp.float32,
                             precision=lax.Precision.HIGHEST))
    out = jnp.maximum(out, 0.0)
    preh_ref[...] = out
    s1 = jnp.sum(out, axis=0)[None, :]
    s2 = jnp.sum(out * out, axis=0)[None, :]

    @pl.when(i == 0)
    def _():
        acc_ref[0:1] = s1
        acc_ref[1:2] = s2

    @pl.when(i > 0)
    def _():
        acc_ref[0:1] += s1
        acc_ref[1:2] += s2

    @pl.when(i == NBLK - 1)
    def _():
        mean = acc_ref[0:1] / N
        var = acc_ref[1:2] / N - mean * mean
        rstd = lax.rsqrt(var + BN_EPS)
        sc = g_ref[...] * rstd
        scale_ref[...] = sc
        shift_ref[...] = be_ref[...] - mean * sc


def _final_body(preh_ref, scale_ref, shift_ref, wp_ref, bp_ref, out_ref):
    hn = preh_ref[...] * scale_ref[...] + shift_ref[...]
    out_ref[...] = lax.dot_general(hn.astype(jnp.bfloat16), wp_ref[...],
                                   (((1,), (0,)), ((), ())),
                                   preferred_element_type=jnp.float32) + bp_ref[...]


_full = lambda shape: pl.BlockSpec(shape, lambda i: tuple(0 for _ in shape))
_rowblk = pl.BlockSpec((BLK, H), lambda i: (i, 0))

_prep_call = pl.pallas_call(
    _prep_body,
    grid=(25,),
    in_specs=[pl.BlockSpec((25, 8, 128), lambda i: (i, 0, 0))] * 2,
    out_specs=[pl.BlockSpec((25, 8, 128), lambda i: (i, 0, 0))] * 2,
    out_shape=[jax.ShapeDtypeStruct((625, 8, 128), jnp.int32)] * 2,
)

_stage_a1_call = pl.pallas_call(
    _stage_a1_body,
    grid=(NBLK,),
    in_specs=[
        _rowblk,
        _full((1, H)),
        _full((1, H)),
        _full((H, R * H)),
    ],
    out_specs=pl.BlockSpec((R, BLK, H), lambda i: (0, i, 0)),
    out_shape=jax.ShapeDtypeStruct((R, N, H), jnp.float32),
)

_stage_a2_call = pl.pallas_call(
    _stage_a2_body,
    grid=(NBLK,),
    in_specs=[
        _rowblk,
        _full((1, H)),
        _full((1, H)),
        _full((H, H)),
        _full((1, H)),
        _rowblk,
    ],
    out_specs=[
        _rowblk,
        _full((H, H)),
    ],
    out_shape=[
        jax.ShapeDtypeStruct((N, H), jnp.float32),
        jax.ShapeDtypeStruct((H, H), jnp.float32),
    ],
)

_stage_c_call = pl.pallas_call(
    _stage_c_body,
    grid=(NBLK,),
    in_specs=[
        _rowblk,
        pl.BlockSpec((2, BLK, HH), lambda i: (0, i, 0)),
        _full((H, H)),
        _full((H, H)),
        _rowblk,
        _full((1, H)),
        _full((1, H)),
    ],
    out_specs=[
        _rowblk,
        _full((1, H)),
        _full((1, H)),
    ],
    out_shape=[
        jax.ShapeDtypeStruct((N, H), jnp.float32),
        jax.ShapeDtypeStruct((1, H), jnp.float32),
        jax.ShapeDtypeStruct((1, H), jnp.float32),
    ],
    scratch_shapes=[pltpu.VMEM((8, H), jnp.float32)],
)

_final_call = pl.pallas_call(
    _final_body,
    grid=(NBLK,),
    in_specs=[
        _rowblk,
        _full((1, H)),
        _full((1, H)),
        _full((H, H)),
        _full((1, H)),
    ],
    out_specs=_rowblk,
    out_shape=jax.ShapeDtypeStruct((N, H), jnp.float32),
)


def _sc_scatter_body(hv_hbm, idx_hbm, dst_hbm, zer_hbm, out_hbm,
                     idx_v, dst_v, rows_v, agg_sh, *sems):
    cid = lax.axis_index("c")
    sid = lax.axis_index("s")
    gsems = sems[:NBUF]
    ssems = sems[NBUF:]

    pltpu.sync_copy(zer_hbm.at[pl.ds(sid * RPT, RPT)],
                    agg_sh.at[pl.ds(sid * RPT, RPT)])
    pltpu.sync_copy(idx_hbm.at[cid, sid], idx_v)
    pltpu.sync_copy(dst_hbm.at[sid], dst_v)
    plsc.subcore_barrier()

    for b in range(GLEAD):
        pltpu.async_copy(hv_hbm.at[idx_v.at[b]], rows_v.at[b], gsems[b])

    def chunk(c, b):
        pltpu.make_async_copy(hv_hbm.at[idx_v.at[c]], rows_v.at[b],
                              gsems[b]).wait()
        pltpu.async_copy(rows_v.at[b], agg_sh.at[dst_v.at[c]], ssems[b],
                         add=True)
        d = (b + GLEAD) % NBUF

        @pl.when(c + GLEAD - NBUF >= 0)
        def _():
            pltpu.make_async_copy(rows_v.at[d],
                                  agg_sh.at[dst_v.at[c + GLEAD - NBUF]],
                                  ssems[d]).wait()

        @pl.when(c + GLEAD < CHUNKS)
        def _():
            pltpu.async_copy(hv_hbm.at[idx_v.at[c + GLEAD]], rows_v.at[d],
                             gsems[d])

    def loop_body(i, carry):
        for b in range(NBUF):
            chunk(i * NBUF + b, b)
        return carry

    lax.fori_loop(0, CHUNKS // NBUF, loop_body, 0)

    for b in range(GLEAD, NBUF):
        pltpu.make_async_copy(rows_v.at[b],
                              agg_sh.at[dst_v.at[CHUNKS - NBUF + b]],
                              ssems[b]).wait()

    plsc.subcore_barrier()
    pltpu.sync_copy(agg_sh.at[pl.ds(sid * RPT, RPT)],
                    out_hbm.at[cid, pl.ds(sid * RPT, RPT)])


_sc_scatter_call = pl.kernel(
    _sc_scatter_body,
    out_type=jax.ShapeDtypeStruct((NC, NP, HH), jnp.float32),
    mesh=plsc.VectorSubcoreMesh(core_axis_name="c", subcore_axis_name="s",
                                num_cores=NC),
    scratch_types=[
        pltpu.VMEM((CHUNKS, K), jnp.int32),
        pltpu.VMEM((CHUNKS, K), jnp.int32),
        pltpu.VMEM((NBUF, K, HH), jnp.float32),
        pltpu.VMEM_SHARED((NP, HH), jnp.float32),
    ] + [pltpu.SemaphoreType.DMA] * (2 * NBUF),
    compiler_params=pltpu.CompilerParams(use_tc_tiling_on_sc=False),
)


def kernel(x, edge_index, edge_attr, batch, A, V, Rw, b, gamma, beta, Wp, bp):
    f32 = jnp.float32
    bf16 = jnp.bfloat16
    src = edge_index[:, 0, :].reshape(625, 8, 128)
    ear = edge_attr.reshape(625, 8, 128)
    idx0, idx1 = _prep_call(src, ear)
    pad = EPTP - EPT

    def _tile_pad(a, padvals):
        a = a.reshape(T, NS, EPT)
        tailv = jnp.broadcast_to(padvals, (T, NS, pad))
        return jnp.concatenate([a, tailv], axis=-1).reshape(T, NS, CHUNKS, K)

    zpad = jnp.zeros((pad,), jnp.int32)
    idx = jnp.stack([_tile_pad(idx0, zpad), _tile_pad(idx1, zpad + 1)],
                    axis=1)
    dpad = N + (jnp.arange(pad, dtype=jnp.int32) % (NP - N))
    dst = _tile_pad(edge_index[:, 1, :], dpad)

    bcols = jnp.arange(H, dtype=batch.dtype)
    Bp = (batch[:, :, None] == bcols[None, None, :]).astype(f32)

    zeros = jnp.zeros((NP, HH), f32)
    ones_r = jnp.ones((1, H), f32)
    zeros_r = jnp.zeros((1, H), f32)

    preh = x
    scale, shift = ones_r, zeros_r
    for t in range(T):
        for l in range(L):
            li = t * L + l
            vcat = jnp.concatenate(
                [V[li, r] for r in range(R)], axis=1).astype(bf16)
            hv = _stage_a1_call(preh, scale, shift, vcat)
            agg = _sc_scatter_call(
                hv.reshape(R * N * NC, HH), idx[t], dst[t], zeros)
            ha, pooled = _stage_a2_call(
                preh, scale, shift, A[li].astype(bf16), b[li][None, :], Bp[t])
            preh, scale, shift = _stage_c_call(
                ha, agg, pooled, Rw[li].astype(bf16), Bp[t],
                gamma[li][None, :], beta[li][None, :])

    wp_pad = jnp.concatenate([Wp, jnp.zeros((H, H - O), f32)],
                             axis=1).astype(bf16)
    bp_pad = jnp.concatenate([bp, jnp.zeros((H - O,), f32)])[None, :]
    out = _final_call(preh, scale, shift, wp_pad, bp_pad)
    return out[:, :O]


# speedup vs baseline: 2.8962x; 2.8962x over previous
import jax
import jax.numpy as jnp
from jax import lax
from jax.experimental import pallas as pl
from jax.experimental.pallas import tpu as pltpu
import jax.experimental.pallas.tpu_sc as plsc

N = 10000
E = 320000
T = 2
L = 2
H = 128
O = 64
R = 5
G = 32
BN_EPS = 1e-5

BLK = 1000
NBLK = N // BLK

NC = 2
NS = 16
HH = H // NC
EPT = E // NS
K = 80
EPTP = EPT
CHUNKS = EPTP // K
NBUF = 5
NP = 10240
RPT = NP // NS


def _prep_body(src_ref, ea_ref, idx0_ref, idx1_ref):
    base = (ea_ref[...] * N + src_ref[...]) * NC
    idx0_ref[...] = base
    idx1_ref[...] = base + 1


def _stage_a1_body(preh_ref, scale_ref, shift_ref, vcat_ref, hv_ref):
    hn = preh_ref[...] * scale_ref[...] + shift_ref[...]
    res = lax.dot_general(hn.astype(jnp.bfloat16), vcat_ref[...],
                          (((1,), (0,)), ((), ())),
                          preferred_element_type=jnp.float32)
    for r in range(R):
        hv_ref[r] = res[:, r * H:(r + 1) * H]


def _stage_a2_body(preh_ref, scale_ref, shift_ref, a_ref, b_ref, bp_ref,
                   ha_ref, pooled_ref):
    i = pl.program_id(0)
    hn = preh_ref[...] * scale_ref[...] + shift_ref[...]
    ha_ref[...] = lax.dot_general(hn.astype(jnp.bfloat16), a_ref[...],
                                  (((1,), (0,)), ((), ())),
                                  preferred_element_type=jnp.float32) + b_ref[...]
    part = lax.dot_general(bp_ref[...], hn, (((0,), (0,)), ((), ())),
                           preferred_element_type=jnp.float32,
                           precision=lax.Precision.HIGHEST)

    @pl.when(i == 0)
    def _():
        pooled_ref[...] = part

    @pl.when(i > 0)
    def _():
        pooled_ref[...] += part


def _stage_c_body(ha_ref, agg_ref, pooled_ref, rw_ref, bp_ref, g_ref, be_ref,
                  preh_ref, scale_ref, shift_ref, acc_ref):
    i = pl.program_id(0)
    rd = lax.dot_general(pooled_ref[...].astype(jnp.bfloat16), rw_ref[...],
                         (((1,), (0,)), ((), ())),
                         preferred_element_type=jnp.float32)
    agg_full = jnp.concatenate([agg_ref[0], agg_ref[1]], axis=1)
    out = (ha_ref[...] + agg_full
           + lax.dot_general(bp_ref[...], rd, (((1,), (0,)), ((), ())),
                             preferred_element_type=jnp.float32,
                             precision=lax.Precision.HIGHEST))
    out = jnp.maximum(out, 0.0)
    preh_ref[...] = out
    s1 = jnp.sum(out, axis=0)[None, :]
    s2 = jnp.sum(out * out, axis=0)[None, :]

    @pl.when(i == 0)
    def _():
        acc_ref[0:1] = s1
        acc_ref[1:2] = s2

    @pl.when(i > 0)
    def _():
        acc_ref[0:1] += s1
        acc_ref[1:2] += s2

    @pl.when(i == NBLK - 1)
    def _():
        mean = acc_ref[0:1] / N
        var = acc_ref[1:2] / N - mean * mean
        rstd = lax.rsqrt(var + BN_EPS)
        sc = g_ref[...] * rstd
        scale_ref[...] = sc
        shift_ref[...] = be_ref[...] - mean * sc


def _final_body(preh_ref, scale_ref, shift_ref, wp_ref, bp_ref, out_ref):
    hn = preh_ref[...] * scale_ref[...] + shift_ref[...]
    out_ref[...] = lax.dot_general(hn.astype(jnp.bfloat16), wp_ref[...],
                                   (((1,), (0,)), ((), ())),
                                   preferred_element_type=jnp.float32) + bp_ref[...]


_full = lambda shape: pl.BlockSpec(shape, lambda i: tuple(0 for _ in shape))
_rowblk = pl.BlockSpec((BLK, H), lambda i: (i, 0))

_prep_call = pl.pallas_call(
    _prep_body,
    grid=(25,),
    in_specs=[pl.BlockSpec((25, 8, 128), lambda i: (i, 0, 0))] * 2,
    out_specs=[pl.BlockSpec((25, 8, 128), lambda i: (i, 0, 0))] * 2,
    out_shape=[jax.ShapeDtypeStruct((625, 8, 128), jnp.int32)] * 2,
)

_stage_a1_call = pl.pallas_call(
    _stage_a1_body,
    grid=(NBLK,),
    in_specs=[
        _rowblk,
        _full((1, H)),
        _full((1, H)),
        _full((H, R * H)),
    ],
    out_specs=pl.BlockSpec((R, BLK, H), lambda i: (0, i, 0)),
    out_shape=jax.ShapeDtypeStruct((R, N, H), jnp.float32),
)

_stage_a2_call = pl.pallas_call(
    _stage_a2_body,
    grid=(NBLK,),
    in_specs=[
        _rowblk,
        _full((1, H)),
        _full((1, H)),
        _full((H, H)),
        _full((1, H)),
        _rowblk,
    ],
    out_specs=[
        _rowblk,
        _full((H, H)),
    ],
    out_shape=[
        jax.ShapeDtypeStruct((N, H), jnp.float32),
        jax.ShapeDtypeStruct((H, H), jnp.float32),
    ],
)

_stage_c_call = pl.pallas_call(
    _stage_c_body,
    grid=(NBLK,),
    in_specs=[
        _rowblk,
        pl.BlockSpec((2, BLK, HH), lambda i: (0, i, 0)),
        _full((H, H)),
        _full((H, H)),
        _rowblk,
        _full((1, H)),
        _full((1, H)),
    ],
    out_specs=[
        _rowblk,
        _full((1, H)),
        _full((1, H)),
    ],
    out_shape=[
        jax.ShapeDtypeStruct((N, H), jnp.float32),
        jax.ShapeDtypeStruct((1, H), jnp.float32),
        jax.ShapeDtypeStruct((1, H), jnp.float32),
    ],
    scratch_shapes=[pltpu.VMEM((8, H), jnp.float32)],
)

_final_call = pl.pallas_call(
    _final_body,
    grid=(NBLK,),
    in_specs=[
        _rowblk,
        _full((1, H)),
        _full((1, H)),
        _full((H, H)),
        _full((1, H)),
    ],
    out_specs=_rowblk,
    out_shape=jax.ShapeDtypeStruct((N, H), jnp.float32),
)


def _sc_scatter_body(hv_hbm, idx_hbm, dst_hbm, zer_hbm, out_hbm,
                     idx_v, dst_v, rows_v, agg_sh, *sems):
    cid = lax.axis_index("c")
    sid = lax.axis_index("s")

    pltpu.sync_copy(zer_hbm.at[pl.ds(sid * RPT, RPT)],
                    agg_sh.at[pl.ds(sid * RPT, RPT)])
    pltpu.sync_copy(idx_hbm.at[cid, sid], idx_v)
    pltpu.sync_copy(dst_hbm.at[sid], dst_v)
    plsc.subcore_barrier()

    for b in range(NBUF):
        pltpu.async_copy(hv_hbm.at[idx_v.at[b]], rows_v.at[b], sems[b])

    def chunk(c, b):
        pltpu.make_async_copy(hv_hbm.at[idx_v.at[c]], rows_v.at[b],
                              sems[b]).wait()
        pltpu.sync_copy(rows_v.at[b], agg_sh.at[dst_v.at[c]], add=True)

        @pl.when(c + NBUF < CHUNKS)
        def _():
            pltpu.async_copy(hv_hbm.at[idx_v.at[c + NBUF]], rows_v.at[b],
                             sems[b])

    def loop_body(i, carry):
        for b in range(NBUF):
            chunk(i * NBUF + b, b)
        return carry

    lax.fori_loop(0, CHUNKS // NBUF, loop_body, 0)

    plsc.subcore_barrier()
    pltpu.sync_copy(agg_sh.at[pl.ds(sid * RPT, RPT)],
                    out_hbm.at[cid, pl.ds(sid * RPT, RPT)])


_sc_scatter_call = pl.kernel(
    _sc_scatter_body,
    out_type=jax.ShapeDtypeStruct((NC, NP, HH), jnp.float32),
    mesh=plsc.VectorSubcoreMesh(core_axis_name="c", subcore_axis_name="s",
                                num_cores=NC),
    scratch_types=[
        pltpu.VMEM((CHUNKS, K), jnp.int32),
        pltpu.VMEM((CHUNKS, K), jnp.int32),
        pltpu.VMEM((NBUF, K, HH), jnp.float32),
        pltpu.VMEM_SHARED((NP, HH), jnp.float32),
    ] + [pltpu.SemaphoreType.DMA] * NBUF,
    compiler_params=pltpu.CompilerParams(use_tc_tiling_on_sc=False),
)


def kernel(x, edge_index, edge_attr, batch, A, V, Rw, b, gamma, beta, Wp, bp):
    f32 = jnp.float32
    bf16 = jnp.bfloat16
    src = edge_index[:, 0, :].reshape(625, 8, 128)
    ear = edge_attr.reshape(625, 8, 128)
    idx0, idx1 = _prep_call(src, ear)

    def _tile_reshape(a):
        return a.reshape(T, NS, CHUNKS, K)

    idx = jnp.stack([_tile_reshape(idx0), _tile_reshape(idx1)],
                    axis=1)
    dst = _tile_reshape(edge_index[:, 1, :])

    bcols = jnp.arange(H, dtype=batch.dtype)
    Bp = (batch[:, :, None] == bcols[None, None, :]).astype(f32)

    zeros = jnp.zeros((NP, HH), f32)
    ones_r = jnp.ones((1, H), f32)
    zeros_r = jnp.zeros((1, H), f32)

    preh = x
    scale, shift = ones_r, zeros_r
    for t in range(T):
        for l in range(L):
            li = t * L + l
            vcat = jnp.concatenate(
                [V[li, r] for r in range(R)], axis=1).astype(bf16)
            hv = _stage_a1_call(preh, scale, shift, vcat)
            agg = _sc_scatter_call(
                hv.reshape(R * N * NC, HH), idx[t], dst[t], zeros)
            ha, pooled = _stage_a2_call(
                preh, scale, shift, A[li].astype(bf16), b[li][None, :], Bp[t])
            preh, scale, shift = _stage_c_call(
                ha, agg, pooled, Rw[li].astype(bf16), Bp[t],
                gamma[li][None, :], beta[li][None, :])

    wp_pad = jnp.concatenate([Wp, jnp.zeros((H, H - O), f32)],
                             axis=1).astype(bf16)
    bp_pad = jnp.concatenate([bp, jnp.zeros((H - O,), f32)])[None, :]
    out = _final_call(preh, scale, shift, wp_pad, bp_pad)
    return out[:, :O]
